# Initial kernel scaffold; baseline (speedup 1.0000x reference)
import jax
import jax.numpy as jnp
from jax.experimental import pallas as pl


def kernel(x, edge_index, edge_attr, batch, W1, a1s, a1d, b1, W2, a2s, a2d, b2, W3, a3s, a3d, b3, l1_W, l1_b, fc1_W, fc1_b, fc2_W, fc2_b):
    raise NotImplementedError("write your pallas kernel here")


# speedup vs baseline: 26.4788x; 26.4788x over previous
import functools

import jax
import jax.numpy as jnp
from jax import lax
from jax.experimental import pallas as pl
from jax.experimental.pallas import tpu as pltpu
from jax.experimental.pallas import tpu_sc as plsc

N = 10000
E = 320000
C = 128
G = 128
NPO = 10112
PADN = N
NC = 2
NS = 16
L = 16

NT = NC * NS
NBLK = 9
BR = 9
BLK = BR * 128
KB = 32
NBAT = BLK // KB
EB = NBLK * BLK
EP = NT * EB
VPB = BLK // L
DR = 80
RPT = NPO // NS


def _sc_body(h_hbm, als_hbm, ald_hbm, src_hbm, dst4_hbm,
             out_hbm, ex_hbm,
             als_v, ald_v, den_v, rowidx, zbuf, srcb, dsta, exb,
             rows0, rows1, den_full, out_sh, sem0, sem1):
    cid = lax.axis_index("c")
    sid = lax.axis_index("s")
    z16 = jnp.zeros((L,), jnp.float32)
    tb = 2 * sid + cid

    pltpu.sync_copy(als_hbm, als_v)
    pltpu.sync_copy(ald_hbm, ald_v)

    def _zd(i, c):
        for cc in range(128 // L):
            den_v[i, pl.ds(cc * L, L)] = z16
        return c
    lax.fori_loop(0, DR, _zd, 0)

    for r in range(4):
        for cc in range(C // L):
            zbuf[r, pl.ds(cc * L, L)] = z16
    iota16 = lax.iota(jnp.int32, L)
    for t in range(DR // L):
        rowidx[t, :] = t * L + iota16

    def _zo(i, c):
        pltpu.sync_copy(zbuf, out_sh.at[pl.ds(sid * RPT + i * 4, 4)])
        return c
    lax.fori_loop(0, RPT // 4, _zo, 0)

    @pl.when(sid == 0)
    def _():
        pltpu.sync_copy(den_v, den_full)
    plsc.subcore_barrier()

    def _pa_blk(blk, c):
        tb2 = 2 * sid + blk // NBLK
        bb = blk % NBLK
        pltpu.sync_copy(src_hbm.at[tb2, bb], srcb)
        pltpu.sync_copy(dst4_hbm.at[tb2, bb], dsta)

        def _pa(i, cc):
            r = i // 8
            k = i % 8
            s16 = srcb[r, pl.ds(k * L, L)]
            d16 = dsta[r, pl.ds(k * L, L)]
            z = (plsc.load_gather(als_v, [s16])
                 + plsc.load_gather(ald_v, [d16]))
            e = jnp.where(z > 0.0, z, 0.2 * z)
            ex = jnp.exp(e)
            exb[r, pl.ds(k * L, L)] = ex
            plsc.addupdate_scatter(
                den_v, [lax.shift_right_logical(d16, 7),
                        lax.bitwise_and(d16, 127)], ex)
            return cc
        lax.fori_loop(0, VPB, _pa, 0)
        pltpu.sync_copy(exb, ex_hbm.at[cid, tb2, bb])
        return c
    lax.fori_loop(0, 2 * NBLK, _pa_blk, 0)

    def _dc(t, c):
        pltpu.sync_copy(den_v.at[pl.ds(t * L, L)],
                        den_full.at[rowidx.at[t]], add=True)
        return c
    lax.fori_loop(0, DR // L, _dc, 0)
    plsc.subcore_barrier()
    pltpu.sync_copy(den_full, den_v)

    def _pb_blk(blk, c):
        pltpu.sync_copy(src_hbm.at[tb, blk], srcb)
        pltpu.sync_copy(dst4_hbm.at[tb, blk], dsta)
        pltpu.sync_copy(ex_hbm.at[cid, tb, blk], exb)
        pltpu.async_copy(
            h_hbm.at[srcb.at[0, pl.ds(0, KB)]], rows0, sem0)
        pltpu.async_copy(
            h_hbm.at[srcb.at[0, pl.ds(KB, KB)]], rows1, sem1)

        def _pb(i, cc):
            for p in range(2):
                b = i * 2 + p
                buf = rows0 if p == 0 else rows1
                sem = sem0 if p == 0 else sem1
                pltpu.make_async_copy(
                    h_hbm.at[srcb.at[b // 4, pl.ds((b % 4) * KB, KB)]],
                    buf, sem).wait()
                for kk in range(KB // L):
                    off = (b % 4) * KB + kk * L
                    d16 = dsta[b // 4, pl.ds(off, L)]
                    ex16 = exb[b // 4, pl.ds(off, L)]
                    den16 = plsc.load_gather(
                        den_v, [lax.shift_right_logical(d16, 7),
                                lax.bitwise_and(d16, 127)])
                    alpha = ex16 / (den16 + 1e-16)
                    for jj in range(L):
                        row = kk * L + jj
                        asp = jnp.full((L,), alpha[jj])
                        for cc2 in range(C // L):
                            buf[row, pl.ds(cc2 * L, L)] = (
                                buf[row, pl.ds(cc2 * L, L)] * asp)
                    pltpu.sync_copy(buf.at[pl.ds(kk * L, L)],
                                    out_sh.at[d16], add=True)
                nxt = jnp.minimum(b + 2, NBAT - 1)
                pltpu.async_copy(
                    h_hbm.at[srcb.at[nxt // 4, pl.ds((nxt % 4) * KB, KB)]],
                    buf, sem)
            return cc
        lax.fori_loop(0, NBAT // 2, _pb, 0)

        q = NBAT - 1
        pltpu.make_async_copy(
            h_hbm.at[srcb.at[q // 4, pl.ds((q % 4) * KB, KB)]],
            rows0, sem0).wait()
        pltpu.make_async_copy(
            h_hbm.at[srcb.at[q // 4, pl.ds((q % 4) * KB, KB)]],
            rows1, sem1).wait()
        return c
    lax.fori_loop(0, NBLK, _pb_blk, 0)

    plsc.subcore_barrier()
    pltpu.sync_copy(out_sh.at[pl.ds(sid * RPT, RPT)],
                    out_hbm.at[cid, pl.ds(sid * RPT, RPT)])


_sc_gat = functools.partial(
    pl.kernel,
    out_type=[
        jax.ShapeDtypeStruct((NC, NPO, C), jnp.float32),
        jax.ShapeDtypeStruct((NC, NT, NBLK, BR, 128), jnp.float32),
    ],
    mesh=plsc.VectorSubcoreMesh(
        core_axis_name="c", subcore_axis_name="s",
        num_cores=NC, num_subcores=NS),
    scratch_types=[
        pltpu.VMEM((NPO,), jnp.float32),
        pltpu.VMEM((NPO,), jnp.float32),
        pltpu.VMEM((DR, 128), jnp.float32),
        pltpu.VMEM((DR // L, L), jnp.int32),
        pltpu.VMEM((4, C), jnp.float32),
        pltpu.VMEM((BR, 128), jnp.int32),
        pltpu.VMEM((BR, 128), jnp.int32),
        pltpu.VMEM((BR, 128), jnp.float32),
        pltpu.VMEM((KB, C), jnp.float32),
        pltpu.VMEM((KB, C), jnp.float32),
        pltpu.VMEM_SHARED((DR, 128), jnp.float32),
        pltpu.VMEM_SHARED((NPO, C), jnp.float32),
        pltpu.SemaphoreType.DMA,
        pltpu.SemaphoreType.DMA,
    ],
    compiler_params=pltpu.CompilerParams(needs_layout_passes=False),
)(_sc_body)


def _tc1_body(x_ref, w_ref, as_ref, ad_ref, h_ref, als_ref, ald_ref):
    h = jnp.dot(x_ref[...], w_ref[...], preferred_element_type=jnp.float32)
    h_ref[...] = h
    als_ref[...] = jnp.sum(h * as_ref[...], axis=1)
    ald_ref[...] = jnp.sum(h * ad_ref[...], axis=1)


def _tc2_body(p_ref, b_ref, w_ref, as_ref, ad_ref,
              xin_ref, h_ref, als_ref, ald_ref):
    xin = p_ref[0] + p_ref[1] + b_ref[...]
    xin_ref[...] = xin
    h = jnp.dot(xin, w_ref[...], preferred_element_type=jnp.float32)
    h_ref[...] = h
    als_ref[...] = jnp.sum(h * as_ref[...], axis=1)
    ald_ref[...] = jnp.sum(h * ad_ref[...], axis=1)


def _tc3_body(p_ref, b_ref, r_ref, w_ref, as_ref, ad_ref,
              h_ref, als_ref, ald_ref):
    xin = p_ref[0] + p_ref[1] + b_ref[...] + r_ref[...]
    h = jnp.dot(xin, w_ref[...], preferred_element_type=jnp.float32)
    h_ref[...] = h
    als_ref[...] = jnp.sum(h * as_ref[...], axis=1)
    ald_ref[...] = jnp.sum(h * ad_ref[...], axis=1)


def _pool_body(p_ref, b3_ref, batch_ref, l1w_ref, l1b_ref,
               f1w_ref, f1b_ref, f2w_ref, f2b_ref, y_ref):
    x3 = p_ref[0] + p_ref[1] + b3_ref[...]
    bt = batch_ref[...]
    gid = lax.broadcasted_iota(jnp.int32, (G, NPO), 0)
    oh = (bt[None, :] == gid).astype(jnp.float32)
    s = jnp.dot(oh, x3, preferred_element_type=jnp.float32)
    cnt = jnp.sum(oh, axis=1)
    pooled = s / jnp.maximum(cnt, 1.0)[:, None]
    h1 = jnp.dot(jnp.maximum(pooled, 0.0), l1w_ref[...],
                 preferred_element_type=jnp.float32) + l1b_ref[...]
    h2 = jnp.dot(jnp.maximum(h1, 0.0), f1w_ref[...],
                 preferred_element_type=jnp.float32) + f1b_ref[...]
    y = jnp.dot(jnp.maximum(h2, 0.0), f2w_ref[...],
                preferred_element_type=jnp.float32) + f2b_ref[...]
    y_ref[...] = y * 0.01


_f32 = jnp.float32
_tc1 = pl.pallas_call(_tc1_body, out_shape=[
    jax.ShapeDtypeStruct((NPO, C), _f32),
    jax.ShapeDtypeStruct((NPO,), _f32),
    jax.ShapeDtypeStruct((NPO,), _f32),
])
_tc2 = pl.pallas_call(_tc2_body, out_shape=[
    jax.ShapeDtypeStruct((NPO, C), _f32),
    jax.ShapeDtypeStruct((NPO, C), _f32),
    jax.ShapeDtypeStruct((NPO,), _f32),
    jax.ShapeDtypeStruct((NPO,), _f32),
])
_tc3 = pl.pallas_call(_tc3_body, out_shape=[
    jax.ShapeDtypeStruct((NPO, C), _f32),
    jax.ShapeDtypeStruct((NPO,), _f32),
    jax.ShapeDtypeStruct((NPO,), _f32),
])
_pool = pl.pallas_call(_pool_body, out_shape=[
    jax.ShapeDtypeStruct((G, 1), _f32),
])


def kernel(x, edge_index, edge_attr, batch,
           W1, a1s, a1d, b1, W2, a2s, a2d, b2, W3, a3s, a3d, b3,
           l1_W, l1_b, fc1_W, fc1_b, fc2_W, fc2_b):
    loop = jnp.arange(N, dtype=jnp.int32)
    npad = EP - (E + N)
    padi = jnp.full((npad,), PADN, jnp.int32)
    src = jnp.concatenate([edge_index[0].astype(jnp.int32), loop, padi])
    src = src.reshape(NT, NBLK, BR, 128)
    dst = jnp.concatenate([edge_index[1].astype(jnp.int32), loop, padi])
    dst4 = dst.reshape(NT, NBLK, BR, 128)
    xp = jnp.zeros((NPO, C), _f32).at[:N].set(x)
    batch_p = jnp.concatenate(
        [batch.astype(jnp.int32), jnp.full((NPO - N,), G, jnp.int32)])
    rs = lambda a: a.reshape(1, C)

    h1, als1, ald1 = _tc1(xp, W1, rs(a1s), rs(a1d))
    out1, _ex1 = _sc_gat(h1, als1, ald1, src, dst4)
    x1, h2, als2, ald2 = _tc2(out1, rs(b1), W2, rs(a2s), rs(a2d))
    out2, _ex2 = _sc_gat(h2, als2, ald2, src, dst4)
    h3, als3, ald3 = _tc3(out2, rs(b2), x1, W3, rs(a3s), rs(a3d))
    out3, _ex3 = _sc_gat(h3, als3, ald3, src, dst4)
    (y,) = _pool(out3, rs(b3), batch_p, l1_W, rs(l1_b),
                 fc1_W, rs(fc1_b), fc2_W, fc2_b.reshape(1, 1))
    return y.ravel()


# speedup vs baseline: 27.2182x; 1.0279x over previous
import functools

import jax
import jax.numpy as jnp
from jax import lax
from jax.experimental import pallas as pl
from jax.experimental.pallas import tpu as pltpu
from jax.experimental.pallas import tpu_sc as plsc

N = 10000
E = 320000
C = 128
G = 128
NPO = 10112
PADN = N
NC = 2
NS = 16
L = 16

NT = NC * NS
NBLK = 9
BR = 9
BLK = BR * 128
NBAT = BLK // L
EB = NBLK * BLK
EP = NT * EB
VPB = BLK // L
DR = 80
RPT = NPO // NS


def _sc_body(h_hbm, als_hbm, ald_hbm, sd_hbm, out_hbm,
             als_v, ald_v, den_v, rowidx, zbuf, sd0, sd1,
             rows0, rows1, scaled0, scaled1,
             den_full, out_sh, g0, g1, s0, s1, st0, st1):
    cid = lax.axis_index("c")
    sid = lax.axis_index("s")
    z16 = jnp.zeros((L,), jnp.float32)
    tb = 2 * sid + cid

    pltpu.sync_copy(als_hbm, als_v)
    pltpu.sync_copy(ald_hbm, ald_v)

    def _zd(i, c):
        for cc in range(128 // L):
            den_v[i, pl.ds(cc * L, L)] = z16
        return c
    lax.fori_loop(0, DR, _zd, 0)

    for r in range(4):
        for cc in range(C // L):
            zbuf[r, pl.ds(cc * L, L)] = z16
    iota16 = lax.iota(jnp.int32, L)
    for t in range(DR // L):
        rowidx[t, :] = t * L + iota16

    def _zo(i, c):
        pltpu.sync_copy(zbuf, out_sh.at[pl.ds(sid * RPT + i * 4, 4)])
        return c
    lax.fori_loop(0, RPT // 4, _zo, 0)

    @pl.when(sid == 0)
    def _():
        pltpu.sync_copy(den_v, den_full)
    plsc.subcore_barrier()

    def _ablk(blk):
        return sd_hbm.at[2 * sid + blk // NBLK, blk % NBLK]

    pltpu.async_copy(_ablk(0), sd0, st0)
    pltpu.async_copy(_ablk(1), sd1, st1)

    def _pa_2blk(i, c):
        for p in range(2):
            blk = 2 * i + p
            sd = sd0 if p == 0 else sd1
            sem = st0 if p == 0 else st1
            pltpu.make_async_copy(_ablk(blk), sd, sem).wait()

            def _pa(j, cc):
                r = j // 8
                k = j % 8
                s16 = sd[r, pl.ds(k * L, L)]
                d16 = sd[BR + r, pl.ds(k * L, L)]
                z = (plsc.load_gather(als_v, [s16])
                     + plsc.load_gather(ald_v, [d16]))
                e = jnp.where(z > 0.0, z, 0.2 * z)
                plsc.addupdate_scatter(
                    den_v, [lax.shift_right_logical(d16, 7),
                            lax.bitwise_and(d16, 127)], jnp.exp(e))
                return cc
            lax.fori_loop(0, VPB, _pa, 0)
            nxt = jnp.minimum(blk + 2, 2 * NBLK - 1)
            pltpu.async_copy(_ablk(nxt), sd, sem)
        return c
    lax.fori_loop(0, NBLK, _pa_2blk, 0)
    pltpu.make_async_copy(_ablk(2 * NBLK - 1), sd0, st0).wait()
    pltpu.make_async_copy(_ablk(2 * NBLK - 1), sd1, st1).wait()

    def _dc(t, c):
        pltpu.sync_copy(den_v.at[pl.ds(t * L, L)],
                        den_full.at[rowidx.at[t]], add=True)
        return c
    lax.fori_loop(0, DR // L, _dc, 0)
    plsc.subcore_barrier()
    pltpu.sync_copy(den_full, den_v)

    def _pb_blk(blk, c):
        pltpu.sync_copy(sd_hbm.at[tb, blk], sd0)

        def _gidx(b):
            return sd0.at[b // 8, pl.ds((b % 8) * L, L)]

        def _batch(b, buf, gsem, sbuf, ssem, first):
            pltpu.make_async_copy(h_hbm.at[_gidx(b)], buf, gsem).wait()
            d16 = sd0[BR + b // 8, pl.ds((b % 8) * L, L)]
            s16 = sd0[b // 8, pl.ds((b % 8) * L, L)]
            z = (plsc.load_gather(als_v, [s16])
                 + plsc.load_gather(ald_v, [d16]))
            e = jnp.where(z > 0.0, z, 0.2 * z)
            ex16 = jnp.exp(e)
            den16 = plsc.load_gather(
                den_v, [lax.shift_right_logical(d16, 7),
                        lax.bitwise_and(d16, 127)])
            alpha = ex16 / (den16 + 1e-16)
            if not first:
                pltpu.make_async_copy(sbuf, out_sh.at[d16], ssem).wait()
            for jj in range(L):
                asp = jnp.full((L,), alpha[jj])
                for cc2 in range(C // L):
                    sbuf[jj, pl.ds(cc2 * L, L)] = (
                        buf[jj, pl.ds(cc2 * L, L)] * asp)
            nxt = jnp.minimum(b + 2, NBAT - 1)
            pltpu.async_copy(h_hbm.at[_gidx(nxt)], buf, gsem)
            pltpu.async_copy(sbuf, out_sh.at[d16], ssem, add=True)

        pltpu.async_copy(h_hbm.at[_gidx(0)], rows0, g0)
        pltpu.async_copy(h_hbm.at[_gidx(1)], rows1, g1)
        _batch(0, rows0, g0, scaled0, s0, True)
        _batch(1, rows1, g1, scaled1, s1, True)

        def _pb(i, cc):
            b = 2 + 2 * i
            _batch(b, rows0, g0, scaled0, s0, False)
            _batch(b + 1, rows1, g1, scaled1, s1, False)
            return cc
        lax.fori_loop(0, (NBAT - 2) // 2, _pb, 0)

        dlast = sd0[2 * BR - 1, pl.ds(128 - L, L)]
        pltpu.make_async_copy(scaled0, out_sh.at[dlast], s0).wait()
        pltpu.make_async_copy(scaled1, out_sh.at[dlast], s1).wait()
        pltpu.make_async_copy(
            h_hbm.at[_gidx(NBAT - 1)], rows0, g0).wait()
        pltpu.make_async_copy(
            h_hbm.at[_gidx(NBAT - 1)], rows1, g1).wait()
        return c
    lax.fori_loop(0, NBLK, _pb_blk, 0)

    plsc.subcore_barrier()
    pltpu.sync_copy(out_sh.at[pl.ds(sid * RPT, RPT)],
                    out_hbm.at[cid, pl.ds(sid * RPT, RPT)])


_sc_gat = functools.partial(
    pl.kernel,
    out_type=jax.ShapeDtypeStruct((NC, NPO, C), jnp.float32),
    mesh=plsc.VectorSubcoreMesh(
        core_axis_name="c", subcore_axis_name="s",
        num_cores=NC, num_subcores=NS),
    scratch_types=[
        pltpu.VMEM((NPO,), jnp.float32),
        pltpu.VMEM((NPO,), jnp.float32),
        pltpu.VMEM((DR, 128), jnp.float32),
        pltpu.VMEM((DR // L, L), jnp.int32),
        pltpu.VMEM((4, C), jnp.float32),
        pltpu.VMEM((2 * BR, 128), jnp.int32),
        pltpu.VMEM((2 * BR, 128), jnp.int32),
        pltpu.VMEM((L, C), jnp.float32),
        pltpu.VMEM((L, C), jnp.float32),
        pltpu.VMEM((L, C), jnp.float32),
        pltpu.VMEM((L, C), jnp.float32),
        pltpu.VMEM_SHARED((DR, 128), jnp.float32),
        pltpu.VMEM_SHARED((NPO, C), jnp.float32),
        pltpu.SemaphoreType.DMA,
        pltpu.SemaphoreType.DMA,
        pltpu.SemaphoreType.DMA,
        pltpu.SemaphoreType.DMA,
        pltpu.SemaphoreType.DMA,
        pltpu.SemaphoreType.DMA,
    ],
    compiler_params=pltpu.CompilerParams(needs_layout_passes=False),
)(_sc_body)


def _tc1_body(x_ref, w_ref, as_ref, ad_ref, h_ref, als_ref, ald_ref):
    h = jnp.dot(x_ref[...], w_ref[...], preferred_element_type=jnp.float32)
    h_ref[...] = h
    als_ref[...] = jnp.sum(h * as_ref[...], axis=1)
    ald_ref[...] = jnp.sum(h * ad_ref[...], axis=1)


def _tc2_body(p_ref, b_ref, w_ref, as_ref, ad_ref,
              xin_ref, h_ref, als_ref, ald_ref):
    xin = p_ref[0] + p_ref[1] + b_ref[...]
    xin_ref[...] = xin
    h = jnp.dot(xin, w_ref[...], preferred_element_type=jnp.float32)
    h_ref[...] = h
    als_ref[...] = jnp.sum(h * as_ref[...], axis=1)
    ald_ref[...] = jnp.sum(h * ad_ref[...], axis=1)


def _tc3_body(p_ref, b_ref, r_ref, w_ref, as_ref, ad_ref,
              h_ref, als_ref, ald_ref):
    xin = p_ref[0] + p_ref[1] + b_ref[...] + r_ref[...]
    h = jnp.dot(xin, w_ref[...], preferred_element_type=jnp.float32)
    h_ref[...] = h
    als_ref[...] = jnp.sum(h * as_ref[...], axis=1)
    ald_ref[...] = jnp.sum(h * ad_ref[...], axis=1)


def _pool_body(p_ref, b3_ref, batch_ref, l1w_ref, l1b_ref,
               f1w_ref, f1b_ref, f2w_ref, f2b_ref, y_ref):
    x3 = p_ref[0] + p_ref[1] + b3_ref[...]
    bt = batch_ref[...]
    gid = lax.broadcasted_iota(jnp.int32, (G, NPO), 0)
    oh = (bt[None, :] == gid).astype(jnp.float32)
    s = jnp.dot(oh, x3, preferred_element_type=jnp.float32)
    cnt = jnp.sum(oh, axis=1)
    pooled = s / jnp.maximum(cnt, 1.0)[:, None]
    h1 = jnp.dot(jnp.maximum(pooled, 0.0), l1w_ref[...],
                 preferred_element_type=jnp.float32) + l1b_ref[...]
    h2 = jnp.dot(jnp.maximum(h1, 0.0), f1w_ref[...],
                 preferred_element_type=jnp.float32) + f1b_ref[...]
    y = jnp.dot(jnp.maximum(h2, 0.0), f2w_ref[...],
                preferred_element_type=jnp.float32) + f2b_ref[...]
    y_ref[...] = y * 0.01


_f32 = jnp.float32
_tc1 = pl.pallas_call(_tc1_body, out_shape=[
    jax.ShapeDtypeStruct((NPO, C), _f32),
    jax.ShapeDtypeStruct((NPO,), _f32),
    jax.ShapeDtypeStruct((NPO,), _f32),
])
_tc2 = pl.pallas_call(_tc2_body, out_shape=[
    jax.ShapeDtypeStruct((NPO, C), _f32),
    jax.ShapeDtypeStruct((NPO, C), _f32),
    jax.ShapeDtypeStruct((NPO,), _f32),
    jax.ShapeDtypeStruct((NPO,), _f32),
])
_tc3 = pl.pallas_call(_tc3_body, out_shape=[
    jax.ShapeDtypeStruct((NPO, C), _f32),
    jax.ShapeDtypeStruct((NPO,), _f32),
    jax.ShapeDtypeStruct((NPO,), _f32),
])
_pool = pl.pallas_call(_pool_body, out_shape=[
    jax.ShapeDtypeStruct((G, 1), _f32),
])


def kernel(x, edge_index, edge_attr, batch,
           W1, a1s, a1d, b1, W2, a2s, a2d, b2, W3, a3s, a3d, b3,
           l1_W, l1_b, fc1_W, fc1_b, fc2_W, fc2_b):
    loop = jnp.arange(N, dtype=jnp.int32)
    npad = EP - (E + N)
    padi = jnp.full((npad,), PADN, jnp.int32)
    src = jnp.concatenate([edge_index[0].astype(jnp.int32), loop, padi])
    dst = jnp.concatenate([edge_index[1].astype(jnp.int32), loop, padi])
    sd = jnp.stack([src.reshape(NT, NBLK, BR, 128),
                    dst.reshape(NT, NBLK, BR, 128)], axis=2)
    sd = sd.reshape(NT, NBLK, 2 * BR, 128)
    xp = jnp.zeros((NPO, C), _f32).at[:N].set(x)
    batch_p = jnp.concatenate(
        [batch.astype(jnp.int32), jnp.full((NPO - N,), G, jnp.int32)])
    rs = lambda a: a.reshape(1, C)

    h1, als1, ald1 = _tc1(xp, W1, rs(a1s), rs(a1d))
    out1 = _sc_gat(h1, als1, ald1, sd)
    x1, h2, als2, ald2 = _tc2(out1, rs(b1), W2, rs(a2s), rs(a2d))
    out2 = _sc_gat(h2, als2, ald2, sd)
    h3, als3, ald3 = _tc3(out2, rs(b2), x1, W3, rs(a3s), rs(a3d))
    out3 = _sc_gat(h3, als3, ald3, sd)
    (y,) = _pool(out3, rs(b3), batch_p, l1_W, rs(l1_b),
                 fc1_W, rs(fc1_b), fc2_W, fc2_b.reshape(1, 1))
    return y.ravel()


# speedup vs baseline: 32.4891x; 1.1937x over previous
import functools

import jax
import jax.numpy as jnp
from jax import lax
from jax.experimental import pallas as pl
from jax.experimental.pallas import tpu as pltpu
from jax.experimental.pallas import tpu_sc as plsc

N = 10000
E = 320000
C = 128
G = 128
NPO = 10112
PADN = N
NC = 2
NS = 16
L = 16

NT = NC * NS
NBLK = 9
BR = 9
BLK = BR * 128
NBAT = BLK // L
EB = NBLK * BLK
EP = NT * EB
VPB = BLK // L
DR = 80
RPT = NPO // NS


def _sc_body(h_hbm, als_hbm, ald_hbm, sd_hbm, out_hbm,
             als_v, ald_v, den_v, rowidx, zbuf, sd0, sd1,
             rows0, rows1, scaled0, scaled1,
             den_full, out_sh, g0, g1, s0, s1, ss0, ss1, ss2, ss3,
             st0, st1):
    cid = lax.axis_index("c")
    sid = lax.axis_index("s")
    z16 = jnp.zeros((L,), jnp.float32)
    tb = 2 * sid + cid

    pltpu.sync_copy(als_hbm, als_v)
    pltpu.sync_copy(ald_hbm, ald_v)

    def _zd(i, c):
        for cc in range(128 // L):
            den_v[i, pl.ds(cc * L, L)] = z16
        return c
    lax.fori_loop(0, DR, _zd, 0)

    for r in range(4):
        for cc in range(C // L):
            zbuf[r, pl.ds(cc * L, L)] = z16
    iota16 = lax.iota(jnp.int32, L)
    for t in range(DR // L):
        rowidx[t, :] = t * L + iota16

    def _zo(i, c):
        pltpu.sync_copy(zbuf, out_sh.at[pl.ds(sid * RPT + i * 4, 4)])
        return c
    lax.fori_loop(0, RPT // 4, _zo, 0)

    @pl.when(sid == 0)
    def _():
        pltpu.sync_copy(den_v, den_full)
    plsc.subcore_barrier()

    def _ablk(blk):
        return sd_hbm.at[2 * sid + blk // NBLK, blk % NBLK]

    pltpu.async_copy(_ablk(0), sd0, st0)
    pltpu.async_copy(_ablk(1), sd1, st1)

    def _pa_2blk(i, c):
        for p in range(2):
            blk = 2 * i + p
            sd = sd0 if p == 0 else sd1
            sem = st0 if p == 0 else st1
            pltpu.make_async_copy(_ablk(blk), sd, sem).wait()

            def _pa(j, cc):
                r = j // 8
                k = j % 8
                s16 = sd[r, pl.ds(k * L, L)]
                d16 = sd[BR + r, pl.ds(k * L, L)]
                z = (plsc.load_gather(als_v, [s16])
                     + plsc.load_gather(ald_v, [d16]))
                e = jnp.where(z > 0.0, z, 0.2 * z)
                plsc.addupdate_scatter(
                    den_v, [lax.shift_right_logical(d16, 7),
                            lax.bitwise_and(d16, 127)], jnp.exp(e))
                return cc
            lax.fori_loop(0, VPB, _pa, 0)
            nxt = jnp.minimum(blk + 2, 2 * NBLK - 1)
            pltpu.async_copy(_ablk(nxt), sd, sem)
        return c
    lax.fori_loop(0, NBLK, _pa_2blk, 0)
    pltpu.make_async_copy(_ablk(2 * NBLK - 1), sd0, st0).wait()
    pltpu.make_async_copy(_ablk(2 * NBLK - 1), sd1, st1).wait()

    def _dc(t, c):
        pltpu.sync_copy(den_v.at[pl.ds(t * L, L)],
                        den_full.at[rowidx.at[t]], add=True)
        return c
    lax.fori_loop(0, DR // L, _dc, 0)
    plsc.subcore_barrier()
    pltpu.sync_copy(den_full, den_v)

    RB = (rows0, rows1, scaled0, scaled1)
    GS = (g0, g1, s0, s1)
    SS = (ss0, ss1, ss2, ss3)

    def _pb_blk(blk, c):
        pltpu.sync_copy(sd_hbm.at[tb, blk], sd0)

        def _gidx(b):
            return sd0.at[b // 8, pl.ds((b % 8) * L, L)]

        def _batch(b, p, has_prev):
            buf = RB[p]
            pltpu.make_async_copy(h_hbm.at[_gidx(b)], buf, GS[p]).wait()
            d16 = sd0[BR + b // 8, pl.ds((b % 8) * L, L)]
            s16 = sd0[b // 8, pl.ds((b % 8) * L, L)]
            z = (plsc.load_gather(als_v, [s16])
                 + plsc.load_gather(ald_v, [d16]))
            e = jnp.where(z > 0.0, z, 0.2 * z)
            ex16 = jnp.exp(e)
            den16 = plsc.load_gather(
                den_v, [lax.shift_right_logical(d16, 7),
                        lax.bitwise_and(d16, 127)])
            alpha = ex16 / (den16 + 1e-16)
            q = (p - 1) % 4
            if has_prev:
                pltpu.make_async_copy(RB[q], out_sh.at[d16],
                                      SS[q]).wait()
                nxt = jnp.minimum(b + 3, NBAT - 1)
                pltpu.async_copy(h_hbm.at[_gidx(nxt)], RB[q], GS[q])
            else:
                pltpu.async_copy(h_hbm.at[_gidx(3)], RB[q], GS[q])
            for jj in range(L):
                asp = jnp.full((L,), alpha[jj])
                for cc2 in range(C // L):
                    buf[jj, pl.ds(cc2 * L, L)] = (
                        buf[jj, pl.ds(cc2 * L, L)] * asp)
            pltpu.async_copy(buf, out_sh.at[d16], SS[p], add=True)

        pltpu.async_copy(h_hbm.at[_gidx(0)], RB[0], GS[0])
        pltpu.async_copy(h_hbm.at[_gidx(1)], RB[1], GS[1])
        pltpu.async_copy(h_hbm.at[_gidx(2)], RB[2], GS[2])
        _batch(0, 0, False)
        _batch(1, 1, True)
        _batch(2, 2, True)
        _batch(3, 3, True)

        def _pb(i, cc):
            b = 4 * i
            for p in range(4):
                _batch(b + p, p, True)
            return cc
        lax.fori_loop(1, NBAT // 4, _pb, 0)

        dlast = sd0[2 * BR - 1, pl.ds(128 - L, L)]
        pltpu.make_async_copy(RB[3], out_sh.at[dlast], SS[3]).wait()
        pltpu.make_async_copy(
            h_hbm.at[_gidx(NBAT - 1)], RB[0], GS[0]).wait()
        pltpu.make_async_copy(
            h_hbm.at[_gidx(NBAT - 1)], RB[1], GS[1]).wait()
        pltpu.make_async_copy(
            h_hbm.at[_gidx(NBAT - 1)], RB[2], GS[2]).wait()
        return c
    lax.fori_loop(0, NBLK, _pb_blk, 0)

    plsc.subcore_barrier()
    pltpu.sync_copy(out_sh.at[pl.ds(sid * RPT, RPT)],
                    out_hbm.at[cid, pl.ds(sid * RPT, RPT)])


_sc_gat = functools.partial(
    pl.kernel,
    out_type=jax.ShapeDtypeStruct((NC, NPO, C), jnp.float32),
    mesh=plsc.VectorSubcoreMesh(
        core_axis_name="c", subcore_axis_name="s",
        num_cores=NC, num_subcores=NS),
    scratch_types=[
        pltpu.VMEM((NPO,), jnp.float32),
        pltpu.VMEM((NPO,), jnp.float32),
        pltpu.VMEM((DR, 128), jnp.float32),
        pltpu.VMEM((DR // L, L), jnp.int32),
        pltpu.VMEM((4, C), jnp.float32),
        pltpu.VMEM((2 * BR, 128), jnp.int32),
        pltpu.VMEM((2 * BR, 128), jnp.int32),
        pltpu.VMEM((L, C), jnp.float32),
        pltpu.VMEM((L, C), jnp.float32),
        pltpu.VMEM((L, C), jnp.float32),
        pltpu.VMEM((L, C), jnp.float32),
        pltpu.VMEM_SHARED((DR, 128), jnp.float32),
        pltpu.VMEM_SHARED((NPO, C), jnp.float32),
        pltpu.SemaphoreType.DMA,
        pltpu.SemaphoreType.DMA,
        pltpu.SemaphoreType.DMA,
        pltpu.SemaphoreType.DMA,
        pltpu.SemaphoreType.DMA,
        pltpu.SemaphoreType.DMA,
        pltpu.SemaphoreType.DMA,
        pltpu.SemaphoreType.DMA,
        pltpu.SemaphoreType.DMA,
        pltpu.SemaphoreType.DMA,
    ],
    compiler_params=pltpu.CompilerParams(needs_layout_passes=False),
)(_sc_body)


def _tc1_body(x_ref, w_ref, as_ref, ad_ref, h_ref, als_ref, ald_ref):
    h = jnp.dot(x_ref[...], w_ref[...], preferred_element_type=jnp.float32)
    h_ref[...] = h
    als_ref[...] = jnp.sum(h * as_ref[...], axis=1)
    ald_ref[...] = jnp.sum(h * ad_ref[...], axis=1)


def _tc2_body(p_ref, b_ref, w_ref, as_ref, ad_ref,
              xin_ref, h_ref, als_ref, ald_ref):
    xin = p_ref[0] + p_ref[1] + b_ref[...]
    xin_ref[...] = xin
    h = jnp.dot(xin, w_ref[...], preferred_element_type=jnp.float32)
    h_ref[...] = h
    als_ref[...] = jnp.sum(h * as_ref[...], axis=1)
    ald_ref[...] = jnp.sum(h * ad_ref[...], axis=1)


def _tc3_body(p_ref, b_ref, r_ref, w_ref, as_ref, ad_ref,
              h_ref, als_ref, ald_ref):
    xin = p_ref[0] + p_ref[1] + b_ref[...] + r_ref[...]
    h = jnp.dot(xin, w_ref[...], preferred_element_type=jnp.float32)
    h_ref[...] = h
    als_ref[...] = jnp.sum(h * as_ref[...], axis=1)
    ald_ref[...] = jnp.sum(h * ad_ref[...], axis=1)


def _pool_body(p_ref, b3_ref, batch_ref, l1w_ref, l1b_ref,
               f1w_ref, f1b_ref, f2w_ref, f2b_ref, y_ref):
    x3 = p_ref[0] + p_ref[1] + b3_ref[...]
    bt = batch_ref[...]
    gid = lax.broadcasted_iota(jnp.int32, (G, NPO), 0)
    oh = (bt[None, :] == gid).astype(jnp.float32)
    s = jnp.dot(oh, x3, preferred_element_type=jnp.float32)
    cnt = jnp.sum(oh, axis=1)
    pooled = s / jnp.maximum(cnt, 1.0)[:, None]
    h1 = jnp.dot(jnp.maximum(pooled, 0.0), l1w_ref[...],
                 preferred_element_type=jnp.float32) + l1b_ref[...]
    h2 = jnp.dot(jnp.maximum(h1, 0.0), f1w_ref[...],
                 preferred_element_type=jnp.float32) + f1b_ref[...]
    y = jnp.dot(jnp.maximum(h2, 0.0), f2w_ref[...],
                preferred_element_type=jnp.float32) + f2b_ref[...]
    y_ref[...] = y * 0.01


_f32 = jnp.float32
_tc1 = pl.pallas_call(_tc1_body, out_shape=[
    jax.ShapeDtypeStruct((NPO, C), _f32),
    jax.ShapeDtypeStruct((NPO,), _f32),
    jax.ShapeDtypeStruct((NPO,), _f32),
])
_tc2 = pl.pallas_call(_tc2_body, out_shape=[
    jax.ShapeDtypeStruct((NPO, C), _f32),
    jax.ShapeDtypeStruct((NPO, C), _f32),
    jax.ShapeDtypeStruct((NPO,), _f32),
    jax.ShapeDtypeStruct((NPO,), _f32),
])
_tc3 = pl.pallas_call(_tc3_body, out_shape=[
    jax.ShapeDtypeStruct((NPO, C), _f32),
    jax.ShapeDtypeStruct((NPO,), _f32),
    jax.ShapeDtypeStruct((NPO,), _f32),
])
_pool = pl.pallas_call(_pool_body, out_shape=[
    jax.ShapeDtypeStruct((G, 1), _f32),
])


def kernel(x, edge_index, edge_attr, batch,
           W1, a1s, a1d, b1, W2, a2s, a2d, b2, W3, a3s, a3d, b3,
           l1_W, l1_b, fc1_W, fc1_b, fc2_W, fc2_b):
    loop = jnp.arange(N, dtype=jnp.int32)
    npad = EP - (E + N)
    padi = jnp.full((npad,), PADN, jnp.int32)
    src = jnp.concatenate([edge_index[0].astype(jnp.int32), loop, padi])
    dst = jnp.concatenate([edge_index[1].astype(jnp.int32), loop, padi])
    sd = jnp.stack([src.reshape(NT, NBLK, BR, 128),
                    dst.reshape(NT, NBLK, BR, 128)], axis=2)
    sd = sd.reshape(NT, NBLK, 2 * BR, 128)
    xp = jnp.zeros((NPO, C), _f32).at[:N].set(x)
    batch_p = jnp.concatenate(
        [batch.astype(jnp.int32), jnp.full((NPO - N,), G, jnp.int32)])
    rs = lambda a: a.reshape(1, C)

    h1, als1, ald1 = _tc1(xp, W1, rs(a1s), rs(a1d))
    out1 = _sc_gat(h1, als1, ald1, sd)
    x1, h2, als2, ald2 = _tc2(out1, rs(b1), W2, rs(a2s), rs(a2d))
    out2 = _sc_gat(h2, als2, ald2, sd)
    h3, als3, ald3 = _tc3(out2, rs(b2), x1, W3, rs(a3s), rs(a3d))
    out3 = _sc_gat(h3, als3, ald3, sd)
    (y,) = _pool(out3, rs(b3), batch_p, l1_W, rs(l1_b),
                 fc1_W, rs(fc1_b), fc2_W, fc2_b.reshape(1, 1))
    return y.ravel()


# speedup vs baseline: 33.3046x; 1.0251x over previous
import functools

import jax
import jax.numpy as jnp
from jax import lax
from jax.experimental import pallas as pl
from jax.experimental.pallas import tpu as pltpu
from jax.experimental.pallas import tpu_sc as plsc

N = 10000
E = 320000
C = 128
G = 128
NPO = 10112
PADN = N
NC = 2
NS = 16
L = 16

NT = NC * NS
NBLK = 9
BR = 9
BLK = BR * 128
NBAT = BLK // L
EB = NBLK * BLK
EP = NT * EB
VPB = BLK // L
DR = 80
RPT = NPO // NS


def _sc_body(h_hbm, als_hbm, ald_hbm, sd_hbm, out_hbm,
             als_v, ald_v, den_v, rowidx, zbuf, sd0, sd1,
             rows0, rows1, scaled0, scaled1,
             den_full, out_sh, g0, g1, s0, s1, ss0, ss1, ss2, ss3,
             st0, st1):
    cid = lax.axis_index("c")
    sid = lax.axis_index("s")
    z16 = jnp.zeros((L,), jnp.float32)
    tb = 2 * sid + cid

    pltpu.sync_copy(als_hbm, als_v)
    pltpu.sync_copy(ald_hbm, ald_v)

    def _zd(i, c):
        for cc in range(128 // L):
            den_v[i, pl.ds(cc * L, L)] = z16
        return c
    lax.fori_loop(0, DR, _zd, 0)

    for r in range(4):
        for cc in range(C // L):
            zbuf[r, pl.ds(cc * L, L)] = z16
    iota16 = lax.iota(jnp.int32, L)
    for t in range(DR // L):
        rowidx[t, :] = t * L + iota16

    def _zo(i, c):
        for p in range(2):
            pltpu.async_copy(
                den_v.at[pl.ds(0, 79)],
                out_sh.at[pl.ds(sid * RPT + (2 * i + p) * 79, 79)],
                st0 if p == 0 else st1)
        return c
    lax.fori_loop(0, 4, _zo, 0)
    for _ in range(4):
        pltpu.make_async_copy(
            den_v.at[pl.ds(0, 79)], out_sh.at[pl.ds(sid * RPT, 79)],
            st0).wait()
        pltpu.make_async_copy(
            den_v.at[pl.ds(0, 79)], out_sh.at[pl.ds(sid * RPT, 79)],
            st1).wait()

    @pl.when(sid == 0)
    def _():
        pltpu.sync_copy(den_v, den_full)
    plsc.subcore_barrier()

    def _ablk(blk):
        return sd_hbm.at[2 * sid + blk // NBLK, blk % NBLK]

    pltpu.async_copy(_ablk(0), sd0, st0)
    pltpu.async_copy(_ablk(1), sd1, st1)

    def _pa_2blk(i, c):
        for p in range(2):
            blk = 2 * i + p
            sd = sd0 if p == 0 else sd1
            sem = st0 if p == 0 else st1
            pltpu.make_async_copy(_ablk(blk), sd, sem).wait()

            def _pa(j, cc):
                r = j // 8
                k = j % 8
                s16 = sd[r, pl.ds(k * L, L)]
                d16 = sd[BR + r, pl.ds(k * L, L)]
                z = (plsc.load_gather(als_v, [s16])
                     + plsc.load_gather(ald_v, [d16]))
                e = jnp.where(z > 0.0, z, 0.2 * z)
                plsc.addupdate_scatter(
                    den_v, [lax.shift_right_logical(d16, 7),
                            lax.bitwise_and(d16, 127)], jnp.exp(e))
                return cc
            lax.fori_loop(0, VPB, _pa, 0)
            nxt = jnp.minimum(blk + 2, 2 * NBLK - 1)
            pltpu.async_copy(_ablk(nxt), sd, sem)
        return c
    lax.fori_loop(0, NBLK, _pa_2blk, 0)
    pltpu.make_async_copy(_ablk(2 * NBLK - 1), sd0, st0).wait()
    pltpu.make_async_copy(_ablk(2 * NBLK - 1), sd1, st1).wait()

    def _dc(t, c):
        pltpu.sync_copy(den_v.at[pl.ds(t * L, L)],
                        den_full.at[rowidx.at[t]], add=True)
        return c
    lax.fori_loop(0, DR // L, _dc, 0)
    plsc.subcore_barrier()
    pltpu.sync_copy(den_full, den_v)

    RB = (rows0, rows1, scaled0, scaled1)
    GS = (g0, g1, s0, s1)
    SS = (ss0, ss1, ss2, ss3)

    def _pb_blk(blk, c):
        pltpu.sync_copy(sd_hbm.at[tb, blk], sd0)

        def _gidx(b):
            return sd0.at[b // 8, pl.ds((b % 8) * L, L)]

        def _batch(b, p, has_prev):
            buf = RB[p]
            pltpu.make_async_copy(h_hbm.at[_gidx(b)], buf, GS[p]).wait()
            d16 = sd0[BR + b // 8, pl.ds((b % 8) * L, L)]
            s16 = sd0[b // 8, pl.ds((b % 8) * L, L)]
            z = (plsc.load_gather(als_v, [s16])
                 + plsc.load_gather(ald_v, [d16]))
            e = jnp.where(z > 0.0, z, 0.2 * z)
            ex16 = jnp.exp(e)
            den16 = plsc.load_gather(
                den_v, [lax.shift_right_logical(d16, 7),
                        lax.bitwise_and(d16, 127)])
            alpha = ex16 / (den16 + 1e-16)
            q = (p - 1) % 4
            if has_prev:
                pltpu.make_async_copy(RB[q], out_sh.at[d16],
                                      SS[q]).wait()
                nxt = jnp.minimum(b + 3, NBAT - 1)
                pltpu.async_copy(h_hbm.at[_gidx(nxt)], RB[q], GS[q])
            else:
                pltpu.async_copy(h_hbm.at[_gidx(3)], RB[q], GS[q])
            for jj in range(L):
                asp = jnp.full((L,), alpha[jj])
                for cc2 in range(C // L):
                    buf[jj, pl.ds(cc2 * L, L)] = (
                        buf[jj, pl.ds(cc2 * L, L)] * asp)
            pltpu.async_copy(buf, out_sh.at[d16], SS[p], add=True)

        pltpu.async_copy(h_hbm.at[_gidx(0)], RB[0], GS[0])
        pltpu.async_copy(h_hbm.at[_gidx(1)], RB[1], GS[1])
        pltpu.async_copy(h_hbm.at[_gidx(2)], RB[2], GS[2])
        _batch(0, 0, False)
        _batch(1, 1, True)
        _batch(2, 2, True)
        _batch(3, 3, True)

        def _pb(i, cc):
            b = 4 * i
            for p in range(4):
                _batch(b + p, p, True)
            return cc
        lax.fori_loop(1, NBAT // 4, _pb, 0)

        dlast = sd0[2 * BR - 1, pl.ds(128 - L, L)]
        pltpu.make_async_copy(RB[3], out_sh.at[dlast], SS[3]).wait()
        pltpu.make_async_copy(
            h_hbm.at[_gidx(NBAT - 1)], RB[0], GS[0]).wait()
        pltpu.make_async_copy(
            h_hbm.at[_gidx(NBAT - 1)], RB[1], GS[1]).wait()
        pltpu.make_async_copy(
            h_hbm.at[_gidx(NBAT - 1)], RB[2], GS[2]).wait()
        return c
    lax.fori_loop(0, NBLK, _pb_blk, 0)

    plsc.subcore_barrier()
    pltpu.sync_copy(out_sh.at[pl.ds(sid * RPT, RPT)],
                    out_hbm.at[cid, pl.ds(sid * RPT, RPT)])


_sc_gat = functools.partial(
    pl.kernel,
    out_type=jax.ShapeDtypeStruct((NC, NPO, C), jnp.float32),
    mesh=plsc.VectorSubcoreMesh(
        core_axis_name="c", subcore_axis_name="s",
        num_cores=NC, num_subcores=NS),
    scratch_types=[
        pltpu.VMEM((NPO,), jnp.float32),
        pltpu.VMEM((NPO,), jnp.float32),
        pltpu.VMEM((DR, 128), jnp.float32),
        pltpu.VMEM((DR // L, L), jnp.int32),
        pltpu.VMEM((4, C), jnp.float32),
        pltpu.VMEM((2 * BR, 128), jnp.int32),
        pltpu.VMEM((2 * BR, 128), jnp.int32),
        pltpu.VMEM((L, C), jnp.float32),
        pltpu.VMEM((L, C), jnp.float32),
        pltpu.VMEM((L, C), jnp.float32),
        pltpu.VMEM((L, C), jnp.float32),
        pltpu.VMEM_SHARED((DR, 128), jnp.float32),
        pltpu.VMEM_SHARED((NPO, C), jnp.float32),
        pltpu.SemaphoreType.DMA,
        pltpu.SemaphoreType.DMA,
        pltpu.SemaphoreType.DMA,
        pltpu.SemaphoreType.DMA,
        pltpu.SemaphoreType.DMA,
        pltpu.SemaphoreType.DMA,
        pltpu.SemaphoreType.DMA,
        pltpu.SemaphoreType.DMA,
        pltpu.SemaphoreType.DMA,
        pltpu.SemaphoreType.DMA,
    ],
    compiler_params=pltpu.CompilerParams(needs_layout_passes=False),
)(_sc_body)


def _tc1_body(x_ref, w_ref, as_ref, ad_ref, h_ref, als_ref, ald_ref):
    h = jnp.dot(x_ref[...], w_ref[...], preferred_element_type=jnp.float32)
    h_ref[...] = h
    als_ref[...] = jnp.sum(h * as_ref[...], axis=1)
    ald_ref[...] = jnp.sum(h * ad_ref[...], axis=1)


def _tc2_body(p_ref, b_ref, w_ref, as_ref, ad_ref,
              xin_ref, h_ref, als_ref, ald_ref):
    xin = p_ref[0] + p_ref[1] + b_ref[...]
    xin_ref[...] = xin
    h = jnp.dot(xin, w_ref[...], preferred_element_type=jnp.float32)
    h_ref[...] = h
    als_ref[...] = jnp.sum(h * as_ref[...], axis=1)
    ald_ref[...] = jnp.sum(h * ad_ref[...], axis=1)


def _tc3_body(p_ref, b_ref, r_ref, w_ref, as_ref, ad_ref,
              h_ref, als_ref, ald_ref):
    xin = p_ref[0] + p_ref[1] + b_ref[...] + r_ref[...]
    h = jnp.dot(xin, w_ref[...], preferred_element_type=jnp.float32)
    h_ref[...] = h
    als_ref[...] = jnp.sum(h * as_ref[...], axis=1)
    ald_ref[...] = jnp.sum(h * ad_ref[...], axis=1)


def _pool_body(p_ref, b3_ref, batch_ref, l1w_ref, l1b_ref,
               f1w_ref, f1b_ref, f2w_ref, f2b_ref, y_ref):
    x3 = p_ref[0] + p_ref[1] + b3_ref[...]
    bt = batch_ref[...]
    gid = lax.broadcasted_iota(jnp.int32, (G, NPO), 0)
    oh = (bt[None, :] == gid).astype(jnp.float32)
    s = jnp.dot(oh, x3, preferred_element_type=jnp.float32)
    cnt = jnp.sum(oh, axis=1)
    pooled = s / jnp.maximum(cnt, 1.0)[:, None]
    h1 = jnp.dot(jnp.maximum(pooled, 0.0), l1w_ref[...],
                 preferred_element_type=jnp.float32) + l1b_ref[...]
    h2 = jnp.dot(jnp.maximum(h1, 0.0), f1w_ref[...],
                 preferred_element_type=jnp.float32) + f1b_ref[...]
    y = jnp.dot(jnp.maximum(h2, 0.0), f2w_ref[...],
                preferred_element_type=jnp.float32) + f2b_ref[...]
    y_ref[...] = y * 0.01


_f32 = jnp.float32
_tc1 = pl.pallas_call(_tc1_body, out_shape=[
    jax.ShapeDtypeStruct((NPO, C), _f32),
    jax.ShapeDtypeStruct((NPO,), _f32),
    jax.ShapeDtypeStruct((NPO,), _f32),
])
_tc2 = pl.pallas_call(_tc2_body, out_shape=[
    jax.ShapeDtypeStruct((NPO, C), _f32),
    jax.ShapeDtypeStruct((NPO, C), _f32),
    jax.ShapeDtypeStruct((NPO,), _f32),
    jax.ShapeDtypeStruct((NPO,), _f32),
])
_tc3 = pl.pallas_call(_tc3_body, out_shape=[
    jax.ShapeDtypeStruct((NPO, C), _f32),
    jax.ShapeDtypeStruct((NPO,), _f32),
    jax.ShapeDtypeStruct((NPO,), _f32),
])
_pool = pl.pallas_call(_pool_body, out_shape=[
    jax.ShapeDtypeStruct((G, 1), _f32),
])


def kernel(x, edge_index, edge_attr, batch,
           W1, a1s, a1d, b1, W2, a2s, a2d, b2, W3, a3s, a3d, b3,
           l1_W, l1_b, fc1_W, fc1_b, fc2_W, fc2_b):
    loop = jnp.arange(N, dtype=jnp.int32)
    npad = EP - (E + N)
    padi = jnp.full((npad,), PADN, jnp.int32)
    src = jnp.concatenate([edge_index[0].astype(jnp.int32), loop, padi])
    dst = jnp.concatenate([edge_index[1].astype(jnp.int32), loop, padi])
    sd = jnp.stack([src.reshape(NT, NBLK, BR, 128),
                    dst.reshape(NT, NBLK, BR, 128)], axis=2)
    sd = sd.reshape(NT, NBLK, 2 * BR, 128)
    xp = jnp.zeros((NPO, C), _f32).at[:N].set(x)
    batch_p = jnp.concatenate(
        [batch.astype(jnp.int32), jnp.full((NPO - N,), G, jnp.int32)])
    rs = lambda a: a.reshape(1, C)

    h1, als1, ald1 = _tc1(xp, W1, rs(a1s), rs(a1d))
    out1 = _sc_gat(h1, als1, ald1, sd)
    x1, h2, als2, ald2 = _tc2(out1, rs(b1), W2, rs(a2s), rs(a2d))
    out2 = _sc_gat(h2, als2, ald2, sd)
    h3, als3, ald3 = _tc3(out2, rs(b2), x1, W3, rs(a3s), rs(a3d))
    out3 = _sc_gat(h3, als3, ald3, sd)
    (y,) = _pool(out3, rs(b3), batch_p, l1_W, rs(l1_b),
                 fc1_W, rs(fc1_b), fc2_W, fc2_b.reshape(1, 1))
    return y.ravel()


# speedup vs baseline: 33.9167x; 1.0184x over previous
import functools

import jax
import jax.numpy as jnp
from jax import lax
from jax.experimental import pallas as pl
from jax.experimental.pallas import tpu as pltpu
from jax.experimental.pallas import tpu_sc as plsc

N = 10000
E = 320000
C = 128
G = 128
NPO = 10112
PADN = N
NC = 2
NS = 16
L = 16

NT = NC * NS
NBLK = 9
BR = 9
BLK = BR * 128
KB2 = 32
NBAT = BLK // KB2
EB = NBLK * BLK
EP = NT * EB
VPB = BLK // L
DR = 80
RPT = NPO // NS


def _sc_body(h_hbm, als_hbm, ald_hbm, sd_hbm, out_hbm,
             als_v, ald_v, den_v, sd0, q0, q1, q2,
             den_full, out_sh, g0, g1, g2, ss0, ss1, ss2, st0, st1):
    cid = lax.axis_index("c")
    sid = lax.axis_index("s")
    z16 = jnp.zeros((L,), jnp.float32)
    tb = 2 * sid + cid

    pltpu.sync_copy(als_hbm, als_v)
    pltpu.sync_copy(ald_hbm, ald_v)

    def _zd(i, c):
        for cc in range(128 // L):
            den_v[i, pl.ds(cc * L, L)] = z16
        return c
    lax.fori_loop(0, DR, _zd, 0)

    iota16 = lax.iota(jnp.int32, L)

    def _zo(i, c):
        for p in range(2):
            pltpu.async_copy(
                den_v.at[pl.ds(0, 79)],
                out_sh.at[pl.ds(sid * RPT + (2 * i + p) * 79, 79)],
                st0 if p == 0 else st1)
        return c
    lax.fori_loop(0, 4, _zo, 0)
    for _ in range(4):
        pltpu.make_async_copy(
            den_v.at[pl.ds(0, 79)], out_sh.at[pl.ds(sid * RPT, 79)],
            st0).wait()
        pltpu.make_async_copy(
            den_v.at[pl.ds(0, 79)], out_sh.at[pl.ds(sid * RPT, 79)],
            st1).wait()

    @pl.when(sid == 0)
    def _():
        pltpu.sync_copy(den_v, den_full)
    plsc.subcore_barrier()

    def _ablk(blk):
        return sd_hbm.at[2 * sid + blk // NBLK, blk % NBLK]

    def _pa_blk(blk, c):
        pltpu.sync_copy(_ablk(blk), sd0)

        def _pa(j, cc):
            r = j // 8
            k = j % 8
            s16 = sd0[r, pl.ds(k * L, L)]
            d16 = sd0[BR + r, pl.ds(k * L, L)]
            z = (plsc.load_gather(als_v, [s16])
                 + plsc.load_gather(ald_v, [d16]))
            e = jnp.where(z > 0.0, z, 0.2 * z)
            plsc.addupdate_scatter(
                den_v, [lax.shift_right_logical(d16, 7),
                        lax.bitwise_and(d16, 127)], jnp.exp(e))
            return cc
        lax.fori_loop(0, VPB, _pa, 0)
        return c
    lax.fori_loop(0, 2 * NBLK, _pa_blk, 0)

    def _dc(t, c):
        pltpu.sync_copy(den_v.at[pl.ds(t * L, L)],
                        den_full.at[t * L + iota16], add=True)
        return c
    lax.fori_loop(0, DR // L, _dc, 0)
    plsc.subcore_barrier()
    pltpu.sync_copy(den_full, den_v)

    RB = (q0, q1, q2)
    GS = (g0, g1, g2)
    SS = (ss0, ss1, ss2)

    def _pb_blk(blk, c):
        pltpu.sync_copy(sd_hbm.at[tb, blk], sd0)

        def _gidx(b):
            return sd0.at[b // 4, pl.ds((b % 4) * KB2, KB2)]

        def _dvec(b, kk):
            return sd0[BR + b // 4, pl.ds((b % 4) * KB2 + kk * L, L)]

        def _svec(b, kk):
            return sd0[b // 4, pl.ds((b % 4) * KB2 + kk * L, L)]

        def _batch(b, p, has_prev):
            buf = RB[p]
            pltpu.make_async_copy(h_hbm.at[_gidx(b)], buf, GS[p]).wait()
            q = (p - 1) % 3
            d0 = _dvec(b, 0)
            if has_prev:
                pltpu.make_async_copy(RB[q].at[pl.ds(0, L)],
                                      out_sh.at[d0], SS[q]).wait()
                pltpu.make_async_copy(RB[q].at[pl.ds(0, L)],
                                      out_sh.at[d0], SS[q]).wait()
                nxt = jnp.minimum(b + 2, NBAT - 1)
                pltpu.async_copy(h_hbm.at[_gidx(nxt)], RB[q], GS[q])
            else:
                pltpu.async_copy(h_hbm.at[_gidx(2)], RB[q], GS[q])
            for kk in range(KB2 // L):
                d16 = _dvec(b, kk)
                s16 = _svec(b, kk)
                z = (plsc.load_gather(als_v, [s16])
                     + plsc.load_gather(ald_v, [d16]))
                e = jnp.where(z > 0.0, z, 0.2 * z)
                ex16 = jnp.exp(e)
                den16 = plsc.load_gather(
                    den_v, [lax.shift_right_logical(d16, 7),
                            lax.bitwise_and(d16, 127)])
                alpha = ex16 / (den16 + 1e-16)
                for jj in range(L):
                    row = kk * L + jj
                    asp = jnp.full((L,), alpha[jj])
                    for cc2 in range(C // L):
                        buf[row, pl.ds(cc2 * L, L)] = (
                            buf[row, pl.ds(cc2 * L, L)] * asp)
                pltpu.async_copy(buf.at[pl.ds(kk * L, L)],
                                 out_sh.at[d16], SS[p], add=True)

        pltpu.async_copy(h_hbm.at[_gidx(0)], RB[0], GS[0])
        pltpu.async_copy(h_hbm.at[_gidx(1)], RB[1], GS[1])
        _batch(0, 0, False)
        _batch(1, 1, True)
        _batch(2, 2, True)

        def _pb(i, cc):
            b = 3 * i
            for p in range(3):
                _batch(b + p, p, True)
            return cc
        lax.fori_loop(1, NBAT // 3, _pb, 0)

        dlast = sd0[2 * BR - 1, pl.ds(128 - L, L)]
        pltpu.make_async_copy(RB[2].at[pl.ds(0, L)],
                              out_sh.at[dlast], SS[2]).wait()
        pltpu.make_async_copy(RB[2].at[pl.ds(0, L)],
                              out_sh.at[dlast], SS[2]).wait()
        pltpu.make_async_copy(
            h_hbm.at[_gidx(NBAT - 1)], RB[0], GS[0]).wait()
        pltpu.make_async_copy(
            h_hbm.at[_gidx(NBAT - 1)], RB[1], GS[1]).wait()
        return c
    lax.fori_loop(0, NBLK, _pb_blk, 0)

    plsc.subcore_barrier()
    pltpu.sync_copy(out_sh.at[pl.ds(sid * RPT, RPT)],
                    out_hbm.at[cid, pl.ds(sid * RPT, RPT)])


_sc_gat = functools.partial(
    pl.kernel,
    out_type=jax.ShapeDtypeStruct((NC, NPO, C), jnp.float32),
    mesh=plsc.VectorSubcoreMesh(
        core_axis_name="c", subcore_axis_name="s",
        num_cores=NC, num_subcores=NS),
    scratch_types=[
        pltpu.VMEM((NPO,), jnp.float32),
        pltpu.VMEM((NPO,), jnp.float32),
        pltpu.VMEM((DR, 128), jnp.float32),
        pltpu.VMEM((2 * BR, 128), jnp.int32),
        pltpu.VMEM((KB2, C), jnp.float32),
        pltpu.VMEM((KB2, C), jnp.float32),
        pltpu.VMEM((KB2, C), jnp.float32),
        pltpu.VMEM_SHARED((DR, 128), jnp.float32),
        pltpu.VMEM_SHARED((NPO, C), jnp.float32),
        pltpu.SemaphoreType.DMA,
        pltpu.SemaphoreType.DMA,
        pltpu.SemaphoreType.DMA,
        pltpu.SemaphoreType.DMA,
        pltpu.SemaphoreType.DMA,
        pltpu.SemaphoreType.DMA,
        pltpu.SemaphoreType.DMA,
        pltpu.SemaphoreType.DMA,
    ],
    compiler_params=pltpu.CompilerParams(needs_layout_passes=False),
)(_sc_body)


def _tc1_body(x_ref, w_ref, as_ref, ad_ref, h_ref, als_ref, ald_ref):
    h = jnp.dot(x_ref[...], w_ref[...], preferred_element_type=jnp.float32)
    h_ref[...] = h
    als_ref[...] = jnp.sum(h * as_ref[...], axis=1)
    ald_ref[...] = jnp.sum(h * ad_ref[...], axis=1)


def _tc2_body(p_ref, b_ref, w_ref, as_ref, ad_ref,
              xin_ref, h_ref, als_ref, ald_ref):
    xin = p_ref[0] + p_ref[1] + b_ref[...]
    xin_ref[...] = xin
    h = jnp.dot(xin, w_ref[...], preferred_element_type=jnp.float32)
    h_ref[...] = h
    als_ref[...] = jnp.sum(h * as_ref[...], axis=1)
    ald_ref[...] = jnp.sum(h * ad_ref[...], axis=1)


def _tc3_body(p_ref, b_ref, r_ref, w_ref, as_ref, ad_ref,
              h_ref, als_ref, ald_ref):
    xin = p_ref[0] + p_ref[1] + b_ref[...] + r_ref[...]
    h = jnp.dot(xin, w_ref[...], preferred_element_type=jnp.float32)
    h_ref[...] = h
    als_ref[...] = jnp.sum(h * as_ref[...], axis=1)
    ald_ref[...] = jnp.sum(h * ad_ref[...], axis=1)


def _pool_body(p_ref, b3_ref, batch_ref, l1w_ref, l1b_ref,
               f1w_ref, f1b_ref, f2w_ref, f2b_ref, y_ref):
    x3 = p_ref[0] + p_ref[1] + b3_ref[...]
    bt = batch_ref[...]
    gid = lax.broadcasted_iota(jnp.int32, (G, NPO), 0)
    oh = (bt[None, :] == gid).astype(jnp.float32)
    s = jnp.dot(oh, x3, preferred_element_type=jnp.float32)
    cnt = jnp.sum(oh, axis=1)
    pooled = s / jnp.maximum(cnt, 1.0)[:, None]
    h1 = jnp.dot(jnp.maximum(pooled, 0.0), l1w_ref[...],
                 preferred_element_type=jnp.float32) + l1b_ref[...]
    h2 = jnp.dot(jnp.maximum(h1, 0.0), f1w_ref[...],
                 preferred_element_type=jnp.float32) + f1b_ref[...]
    y = jnp.dot(jnp.maximum(h2, 0.0), f2w_ref[...],
                preferred_element_type=jnp.float32) + f2b_ref[...]
    y_ref[...] = y * 0.01


_f32 = jnp.float32
_tc1 = pl.pallas_call(_tc1_body, out_shape=[
    jax.ShapeDtypeStruct((NPO, C), _f32),
    jax.ShapeDtypeStruct((NPO,), _f32),
    jax.ShapeDtypeStruct((NPO,), _f32),
])
_tc2 = pl.pallas_call(_tc2_body, out_shape=[
    jax.ShapeDtypeStruct((NPO, C), _f32),
    jax.ShapeDtypeStruct((NPO, C), _f32),
    jax.ShapeDtypeStruct((NPO,), _f32),
    jax.ShapeDtypeStruct((NPO,), _f32),
])
_tc3 = pl.pallas_call(_tc3_body, out_shape=[
    jax.ShapeDtypeStruct((NPO, C), _f32),
    jax.ShapeDtypeStruct((NPO,), _f32),
    jax.ShapeDtypeStruct((NPO,), _f32),
])
_pool = pl.pallas_call(_pool_body, out_shape=[
    jax.ShapeDtypeStruct((G, 1), _f32),
])


def kernel(x, edge_index, edge_attr, batch,
           W1, a1s, a1d, b1, W2, a2s, a2d, b2, W3, a3s, a3d, b3,
           l1_W, l1_b, fc1_W, fc1_b, fc2_W, fc2_b):
    loop = jnp.arange(N, dtype=jnp.int32)
    npad = EP - (E + N)
    padi = jnp.full((npad,), PADN, jnp.int32)
    src = jnp.concatenate([edge_index[0].astype(jnp.int32), loop, padi])
    dst = jnp.concatenate([edge_index[1].astype(jnp.int32), loop, padi])
    sd = jnp.stack([src.reshape(NT, NBLK, BR, 128),
                    dst.reshape(NT, NBLK, BR, 128)], axis=2)
    sd = sd.reshape(NT, NBLK, 2 * BR, 128)
    xp = jnp.zeros((NPO, C), _f32).at[:N].set(x)
    batch_p = jnp.concatenate(
        [batch.astype(jnp.int32), jnp.full((NPO - N,), G, jnp.int32)])
    rs = lambda a: a.reshape(1, C)

    h1, als1, ald1 = _tc1(xp, W1, rs(a1s), rs(a1d))
    out1 = _sc_gat(h1, als1, ald1, sd)
    x1, h2, als2, ald2 = _tc2(out1, rs(b1), W2, rs(a2s), rs(a2d))
    out2 = _sc_gat(h2, als2, ald2, sd)
    h3, als3, ald3 = _tc3(out2, rs(b2), x1, W3, rs(a3s), rs(a3d))
    out3 = _sc_gat(h3, als3, ald3, sd)
    (y,) = _pool(out3, rs(b3), batch_p, l1_W, rs(l1_b),
                 fc1_W, rs(fc1_b), fc2_W, fc2_b.reshape(1, 1))
    return y.ravel()


# speedup vs baseline: 33.9580x; 1.0012x over previous
import functools

import jax
import jax.numpy as jnp
from jax import lax
from jax.experimental import pallas as pl
from jax.experimental.pallas import tpu as pltpu
from jax.experimental.pallas import tpu_sc as plsc

N = 10000
E = 320000
C = 128
G = 128
NPO = 10112
PADN = N
NC = 2
NS = 16
L = 16

NT = NC * NS
NBLK = 9
BR = 9
BLK = BR * 128
KB2 = 32
NBAT = BLK // KB2
EB = NBLK * BLK
EP = NT * EB
VPB = BLK // L
DR = 80
RPT = NPO // NS


def _sc_body(h_hbm, als_hbm, ald_hbm, sd_hbm, out_hbm,
             als_v, ald_v, den_v, sd0, q0, q1, q2,
             den_full, out_sh, g0, g1, g2, ss0, ss1, ss2, st0, st1):
    cid = lax.axis_index("c")
    sid = lax.axis_index("s")
    z16 = jnp.zeros((L,), jnp.float32)
    tb = 2 * sid + cid

    pltpu.sync_copy(als_hbm, als_v)
    pltpu.sync_copy(ald_hbm, ald_v)

    def _zd(i, c):
        for cc in range(128 // L):
            den_v[i, pl.ds(cc * L, L)] = z16
        return c
    lax.fori_loop(0, DR, _zd, 0)

    iota16 = lax.iota(jnp.int32, L)

    def _zo(i, c):
        for p in range(2):
            pltpu.async_copy(
                den_v.at[pl.ds(0, 79)],
                out_sh.at[pl.ds(sid * RPT + (2 * i + p) * 79, 79)],
                st0 if p == 0 else st1)
        return c
    lax.fori_loop(0, 4, _zo, 0)
    for _ in range(4):
        pltpu.make_async_copy(
            den_v.at[pl.ds(0, 79)], out_sh.at[pl.ds(sid * RPT, 79)],
            st0).wait()
        pltpu.make_async_copy(
            den_v.at[pl.ds(0, 79)], out_sh.at[pl.ds(sid * RPT, 79)],
            st1).wait()

    @pl.when(sid == 0)
    def _():
        pltpu.sync_copy(den_v, den_full)
    plsc.subcore_barrier()

    def _ablk(blk):
        return sd_hbm.at[2 * sid + blk // NBLK, blk % NBLK]

    def _pa_blk(blk, c):
        pltpu.sync_copy(_ablk(blk), sd0)

        def _pa(j, cc):
            r = j // 8
            k = j % 8
            s16 = sd0[r, pl.ds(k * L, L)]
            d16 = sd0[BR + r, pl.ds(k * L, L)]
            z = (plsc.load_gather(als_v, [s16])
                 + plsc.load_gather(ald_v, [d16]))
            e = jnp.where(z > 0.0, z, 0.2 * z)
            plsc.addupdate_scatter(
                den_v, [lax.shift_right_logical(d16, 7),
                        lax.bitwise_and(d16, 127)], jnp.exp(e))
            return cc
        lax.fori_loop(0, VPB, _pa, 0, unroll=4)
        return c
    lax.fori_loop(0, 2 * NBLK, _pa_blk, 0)

    def _dc(t, c):
        pltpu.sync_copy(den_v.at[pl.ds(t * L, L)],
                        den_full.at[t * L + iota16], add=True)
        return c
    lax.fori_loop(0, DR // L, _dc, 0)
    plsc.subcore_barrier()
    pltpu.sync_copy(den_full, den_v)

    RB = (q0, q1, q2)
    GS = (g0, g1, g2)
    SS = (ss0, ss1, ss2)

    def _pb_blk(blk, c):
        pltpu.sync_copy(sd_hbm.at[tb, blk], sd0)

        def _gidx(b):
            return sd0.at[b // 4, pl.ds((b % 4) * KB2, KB2)]

        def _dvec(b, kk):
            return sd0[BR + b // 4, pl.ds((b % 4) * KB2 + kk * L, L)]

        def _svec(b, kk):
            return sd0[b // 4, pl.ds((b % 4) * KB2 + kk * L, L)]

        def _batch(b, p, has_prev):
            buf = RB[p]
            pltpu.make_async_copy(h_hbm.at[_gidx(b)], buf, GS[p]).wait()
            q = (p - 1) % 3
            d0 = _dvec(b, 0)
            if has_prev:
                pltpu.make_async_copy(RB[q].at[pl.ds(0, L)],
                                      out_sh.at[d0], SS[q]).wait()
                pltpu.make_async_copy(RB[q].at[pl.ds(0, L)],
                                      out_sh.at[d0], SS[q]).wait()
                nxt = jnp.minimum(b + 2, NBAT - 1)
                pltpu.async_copy(h_hbm.at[_gidx(nxt)], RB[q], GS[q])
            else:
                pltpu.async_copy(h_hbm.at[_gidx(2)], RB[q], GS[q])
            for kk in range(KB2 // L):
                d16 = _dvec(b, kk)
                s16 = _svec(b, kk)
                z = (plsc.load_gather(als_v, [s16])
                     + plsc.load_gather(ald_v, [d16]))
                e = jnp.where(z > 0.0, z, 0.2 * z)
                ex16 = jnp.exp(e)
                den16 = plsc.load_gather(
                    den_v, [lax.shift_right_logical(d16, 7),
                            lax.bitwise_and(d16, 127)])
                alpha = ex16 / (den16 + 1e-16)
                for jj in range(L):
                    row = kk * L + jj
                    asp = jnp.full((L,), alpha[jj])
                    for cc2 in range(C // L):
                        buf[row, pl.ds(cc2 * L, L)] = (
                            buf[row, pl.ds(cc2 * L, L)] * asp)
                pltpu.async_copy(buf.at[pl.ds(kk * L, L)],
                                 out_sh.at[d16], SS[p], add=True)

        pltpu.async_copy(h_hbm.at[_gidx(0)], RB[0], GS[0])
        pltpu.async_copy(h_hbm.at[_gidx(1)], RB[1], GS[1])
        _batch(0, 0, False)
        _batch(1, 1, True)
        _batch(2, 2, True)

        def _pb(i, cc):
            b = 3 * i
            for p in range(3):
                _batch(b + p, p, True)
            return cc
        lax.fori_loop(1, NBAT // 3, _pb, 0)

        dlast = sd0[2 * BR - 1, pl.ds(128 - L, L)]
        pltpu.make_async_copy(RB[2].at[pl.ds(0, L)],
                              out_sh.at[dlast], SS[2]).wait()
        pltpu.make_async_copy(RB[2].at[pl.ds(0, L)],
                              out_sh.at[dlast], SS[2]).wait()
        pltpu.make_async_copy(
            h_hbm.at[_gidx(NBAT - 1)], RB[0], GS[0]).wait()
        pltpu.make_async_copy(
            h_hbm.at[_gidx(NBAT - 1)], RB[1], GS[1]).wait()
        return c
    lax.fori_loop(0, NBLK, _pb_blk, 0)

    plsc.subcore_barrier()
    pltpu.sync_copy(out_sh.at[pl.ds(sid * RPT, RPT)],
                    out_hbm.at[cid, pl.ds(sid * RPT, RPT)])


_sc_gat = functools.partial(
    pl.kernel,
    out_type=jax.ShapeDtypeStruct((NC, NPO, C), jnp.float32),
    mesh=plsc.VectorSubcoreMesh(
        core_axis_name="c", subcore_axis_name="s",
        num_cores=NC, num_subcores=NS),
    scratch_types=[
        pltpu.VMEM((NPO,), jnp.float32),
        pltpu.VMEM((NPO,), jnp.float32),
        pltpu.VMEM((DR, 128), jnp.float32),
        pltpu.VMEM((2 * BR, 128), jnp.int32),
        pltpu.VMEM((KB2, C), jnp.float32),
        pltpu.VMEM((KB2, C), jnp.float32),
        pltpu.VMEM((KB2, C), jnp.float32),
        pltpu.VMEM_SHARED((DR, 128), jnp.float32),
        pltpu.VMEM_SHARED((NPO, C), jnp.float32),
        pltpu.SemaphoreType.DMA,
        pltpu.SemaphoreType.DMA,
        pltpu.SemaphoreType.DMA,
        pltpu.SemaphoreType.DMA,
        pltpu.SemaphoreType.DMA,
        pltpu.SemaphoreType.DMA,
        pltpu.SemaphoreType.DMA,
        pltpu.SemaphoreType.DMA,
    ],
    compiler_params=pltpu.CompilerParams(needs_layout_passes=False),
)(_sc_body)


def _tc1_body(x_ref, w_ref, as_ref, ad_ref, h_ref, als_ref, ald_ref):
    h = jnp.dot(x_ref[...], w_ref[...], preferred_element_type=jnp.float32)
    h_ref[...] = h
    als_ref[...] = jnp.sum(h * as_ref[...], axis=1)
    ald_ref[...] = jnp.sum(h * ad_ref[...], axis=1)


def _tc2_body(p_ref, b_ref, w_ref, as_ref, ad_ref,
              xin_ref, h_ref, als_ref, ald_ref):
    xin = p_ref[0] + p_ref[1] + b_ref[...]
    xin_ref[...] = xin
    h = jnp.dot(xin, w_ref[...], preferred_element_type=jnp.float32)
    h_ref[...] = h
    als_ref[...] = jnp.sum(h * as_ref[...], axis=1)
    ald_ref[...] = jnp.sum(h * ad_ref[...], axis=1)


def _tc3_body(p_ref, b_ref, r_ref, w_ref, as_ref, ad_ref,
              h_ref, als_ref, ald_ref):
    xin = p_ref[0] + p_ref[1] + b_ref[...] + r_ref[...]
    h = jnp.dot(xin, w_ref[...], preferred_element_type=jnp.float32)
    h_ref[...] = h
    als_ref[...] = jnp.sum(h * as_ref[...], axis=1)
    ald_ref[...] = jnp.sum(h * ad_ref[...], axis=1)


def _pool_body(p_ref, b3_ref, batch_ref, l1w_ref, l1b_ref,
               f1w_ref, f1b_ref, f2w_ref, f2b_ref, y_ref):
    x3 = p_ref[0] + p_ref[1] + b3_ref[...]
    bt = batch_ref[...]
    gid = lax.broadcasted_iota(jnp.int32, (G, NPO), 0)
    oh = (bt[None, :] == gid).astype(jnp.float32)
    s = jnp.dot(oh, x3, preferred_element_type=jnp.float32)
    cnt = jnp.sum(oh, axis=1)
    pooled = s / jnp.maximum(cnt, 1.0)[:, None]
    h1 = jnp.dot(jnp.maximum(pooled, 0.0), l1w_ref[...],
                 preferred_element_type=jnp.float32) + l1b_ref[...]
    h2 = jnp.dot(jnp.maximum(h1, 0.0), f1w_ref[...],
                 preferred_element_type=jnp.float32) + f1b_ref[...]
    y = jnp.dot(jnp.maximum(h2, 0.0), f2w_ref[...],
                preferred_element_type=jnp.float32) + f2b_ref[...]
    y_ref[...] = y * 0.01


_f32 = jnp.float32
_tc1 = pl.pallas_call(_tc1_body, out_shape=[
    jax.ShapeDtypeStruct((NPO, C), _f32),
    jax.ShapeDtypeStruct((NPO,), _f32),
    jax.ShapeDtypeStruct((NPO,), _f32),
])
_tc2 = pl.pallas_call(_tc2_body, out_shape=[
    jax.ShapeDtypeStruct((NPO, C), _f32),
    jax.ShapeDtypeStruct((NPO, C), _f32),
    jax.ShapeDtypeStruct((NPO,), _f32),
    jax.ShapeDtypeStruct((NPO,), _f32),
])
_tc3 = pl.pallas_call(_tc3_body, out_shape=[
    jax.ShapeDtypeStruct((NPO, C), _f32),
    jax.ShapeDtypeStruct((NPO,), _f32),
    jax.ShapeDtypeStruct((NPO,), _f32),
])
_pool = pl.pallas_call(_pool_body, out_shape=[
    jax.ShapeDtypeStruct((G, 1), _f32),
])


def kernel(x, edge_index, edge_attr, batch,
           W1, a1s, a1d, b1, W2, a2s, a2d, b2, W3, a3s, a3d, b3,
           l1_W, l1_b, fc1_W, fc1_b, fc2_W, fc2_b):
    loop = jnp.arange(N, dtype=jnp.int32)
    npad = EP - (E + N)
    padi = jnp.full((npad,), PADN, jnp.int32)
    src = jnp.concatenate([edge_index[0].astype(jnp.int32), loop, padi])
    dst = jnp.concatenate([edge_index[1].astype(jnp.int32), loop, padi])
    sd = jnp.stack([src.reshape(NT, NBLK, BR, 128),
                    dst.reshape(NT, NBLK, BR, 128)], axis=2)
    sd = sd.reshape(NT, NBLK, 2 * BR, 128)
    xp = jnp.zeros((NPO, C), _f32).at[:N].set(x)
    batch_p = jnp.concatenate(
        [batch.astype(jnp.int32), jnp.full((NPO - N,), G, jnp.int32)])
    rs = lambda a: a.reshape(1, C)

    h1, als1, ald1 = _tc1(xp, W1, rs(a1s), rs(a1d))
    out1 = _sc_gat(h1, als1, ald1, sd)
    x1, h2, als2, ald2 = _tc2(out1, rs(b1), W2, rs(a2s), rs(a2d))
    out2 = _sc_gat(h2, als2, ald2, sd)
    h3, als3, ald3 = _tc3(out2, rs(b2), x1, W3, rs(a3s), rs(a3d))
    out3 = _sc_gat(h3, als3, ald3, sd)
    (y,) = _pool(out3, rs(b3), batch_p, l1_W, rs(l1_b),
                 fc1_W, rs(fc1_b), fc2_W, fc2_b.reshape(1, 1))
    return y.ravel()
